# baseline (speedup 1.0000x reference)
import functools

import jax
import jax.numpy as jnp
from jax import lax
from jax.experimental import pallas as pl
from jax.experimental.pallas import tpu as pltpu
from jax.experimental.pallas import tpu_sc as plsc

_B, _C, _T = 16384, 14, 200
_NB = 8


def kernel(chromatogram_batch, perm):
    x = chromatogram_batch
    p = perm.astype(jnp.int32)
    cmap = jnp.concatenate([
        p,
        jnp.array([6], jnp.int32),
        p + 7,
        jnp.array([13], jnp.int32),
        jnp.array([0, 0], jnp.int32),
    ])

    info = plsc.get_sparse_core_info()
    nc, ns = info.num_cores, info.num_subcores
    nw = nc * ns
    bw = _B // nw
    npairs = bw // (2 * _NB)
    mesh = plsc.VectorSubcoreMesh(core_axis_name="c", subcore_axis_name="s")

    @functools.partial(
        pl.kernel,
        mesh=mesh,
        out_type=jax.ShapeDtypeStruct((_B, _C, _T), jnp.float32),
        compiler_params=pltpu.CompilerParams(needs_layout_passes=False),
        scratch_types=[
            pltpu.VMEM((16,), jnp.int32),
            pltpu.VMEM((_NB, _C, _T), jnp.float32),
            pltpu.VMEM((_NB, _C, _T), jnp.float32),
            pltpu.SemaphoreType.DMA,
            pltpu.SemaphoreType.DMA,
            pltpu.SemaphoreType.DMA,
            pltpu.SemaphoreType.DMA,
        ],
    )
    def k(x_hbm, cmap_hbm, out_hbm, cmap_v, buf_a, buf_b, gsem_a, gsem_b,
          wsem_a, wsem_b):
        wid = lax.axis_index("s") * nc + lax.axis_index("c")
        b0 = wid * bw
        pltpu.sync_copy(cmap_hbm, cmap_v)
        cmapv = cmap_v[...]
        lane = lax.broadcasted_iota(jnp.int32, (16,), 0)
        srcs = [
            jnp.sum(jnp.where(lane == c, cmapv, 0), axis=0) for c in range(_C)
        ]

        def gathers(base, buf, sem):
            return [
                pltpu.async_copy(
                    x_hbm.at[pl.ds(base, _NB), pl.ds(srcs[c], 1)],
                    buf.at[pl.ds(0, _NB), pl.ds(c, 1)],
                    sem,
                )
                for c in range(_C)
            ]

        def drain_write(buf, wsem):
            pltpu.make_async_copy(x_hbm.at[pl.ds(0, _NB)], buf, wsem).wait()

        def do_pair(base, first):
            ga = gathers(base, buf_a, gsem_a)
            if not first:
                drain_write(buf_b, wsem_b)
            gb = gathers(base + _NB, buf_b, gsem_b)
            for cp in ga:
                cp.wait()
            pltpu.async_copy(buf_a, out_hbm.at[pl.ds(base, _NB)], wsem_a)
            for cp in gb:
                cp.wait()
            pltpu.async_copy(buf_b, out_hbm.at[pl.ds(base + _NB, _NB)], wsem_b)

        do_pair(b0, True)

        def body(g, carry):
            base = b0 + g * (2 * _NB)
            drain_write(buf_a, wsem_a)
            do_pair(base, False)
            return carry

        lax.fori_loop(1, npairs, body, 0)
        drain_write(buf_a, wsem_a)
        drain_write(buf_b, wsem_b)

    return k(x, cmap)
